# baseline (speedup 1.0000x reference)
import functools

import jax
import jax.numpy as jnp
from jax import lax
from jax.experimental import pallas as pl
from jax.experimental.pallas import tpu as pltpu
from jax.experimental.pallas import tpu_sc as plsc

_INFO = plsc.get_sparse_core_info()
_NC, _NS = _INFO.num_cores, _INFO.num_subcores
_NW = _NC * _NS


@functools.lru_cache(maxsize=None)
def _build(vocab, n_embd, n_tokens):
    assert n_tokens % _NW == 0
    per_w = n_tokens // _NW
    chunk = 128
    assert per_w % chunk == 0
    n_chunks = per_w // chunk

    mesh = plsc.VectorSubcoreMesh(core_axis_name="c", subcore_axis_name="s")

    @functools.partial(
        pl.kernel,
        out_type=jax.ShapeDtypeStruct((n_tokens, n_embd), jnp.float32),
        mesh=mesh,
        scratch_types=[
            pltpu.VMEM((n_chunks, chunk), jnp.int32),
            pltpu.VMEM((chunk, n_embd), jnp.float32),
            pltpu.SemaphoreType.DMA,
        ],
        compiler_params=pltpu.CompilerParams(use_tc_tiling_on_sc=False),
    )
    def emb(table_hbm, idx_hbm, out_hbm, idx_v, rows_v, sem):
        wid = lax.axis_index("s") * _NC + lax.axis_index("c")
        base = wid * per_w
        pltpu.sync_copy(idx_hbm.at[wid], idx_v)

        @pl.loop(0, n_chunks)
        def _(j):
            pltpu.async_copy(table_hbm.at[idx_v.at[j]], rows_v, sem).wait()
            pltpu.sync_copy(rows_v, out_hbm.at[pl.ds(base + j * chunk, chunk)])

    return emb, per_w, chunk


def kernel(token_ids, emb_weight):
    b, t = token_ids.shape
    vocab, n_embd = emb_weight.shape
    n_tokens = b * t
    emb, _, chunk = _build(vocab, n_embd, n_tokens)
    idx = token_ids.astype(jnp.int32).reshape(_NW, -1, chunk)
    out = emb(emb_weight, idx)
    return out.reshape(b, t, n_embd)


# speedup vs baseline: 1.1141x; 1.1141x over previous
import functools

import jax
import jax.numpy as jnp
from jax import lax
from jax.experimental import pallas as pl
from jax.experimental.pallas import tpu as pltpu
from jax.experimental.pallas import tpu_sc as plsc

_INFO = plsc.get_sparse_core_info()
_NC, _NS = _INFO.num_cores, _INFO.num_subcores
_NW = _NC * _NS


@functools.lru_cache(maxsize=None)
def _build(vocab, n_embd, n_tokens):
    assert n_tokens % _NW == 0
    per_w = n_tokens // _NW
    chunk = 128
    assert per_w % chunk == 0
    n_chunks = per_w // chunk

    nbuf = 8
    assert n_chunks % nbuf == 0

    mesh = plsc.VectorSubcoreMesh(core_axis_name="c", subcore_axis_name="s")

    @functools.partial(
        pl.kernel,
        out_type=jax.ShapeDtypeStruct((n_tokens, n_embd), jnp.float32),
        mesh=mesh,
        scratch_types=[
            pltpu.VMEM((n_chunks, chunk), jnp.int32),
            pltpu.VMEM((nbuf, chunk, n_embd), jnp.float32),
        ]
        + [pltpu.SemaphoreType.DMA] * nbuf,
        compiler_params=pltpu.CompilerParams(use_tc_tiling_on_sc=False),
    )
    def emb(table_hbm, idx_hbm, out_hbm, idx_v, bufs, *sems):
        wid = lax.axis_index("s") * _NC + lax.axis_index("c")
        base = wid * per_w
        pltpu.sync_copy(idx_hbm.at[wid], idx_v)

        def start(j, b):
            pltpu.async_copy(table_hbm.at[idx_v.at[j]], bufs.at[b], sems[b])

        def wait(b):
            pltpu.make_async_copy(
                table_hbm.at[pl.ds(0, chunk)], bufs.at[b], sems[b]
            ).wait()

        for b in range(nbuf):
            start(b, b)

        @pl.loop(0, n_chunks, step=nbuf)
        def _(j0):
            for b in range(nbuf):
                j = j0 + b
                wait(b)
                pltpu.sync_copy(
                    bufs.at[b], out_hbm.at[pl.ds(base + j * chunk, chunk)]
                )
                nxt = j + nbuf

                @pl.when(nxt < n_chunks)
                def _():
                    start(nxt, b)

    return emb, per_w, chunk


def kernel(token_ids, emb_weight):
    b, t = token_ids.shape
    vocab, n_embd = emb_weight.shape
    n_tokens = b * t
    emb, _, chunk = _build(vocab, n_embd, n_tokens)
    idx = token_ids.astype(jnp.int32).reshape(_NW, -1, chunk)
    out = emb(emb_weight, idx)
    return out.reshape(b, t, n_embd)


# speedup vs baseline: 1.4816x; 1.3299x over previous
import functools

import jax
import jax.numpy as jnp
from jax import lax
from jax.experimental import pallas as pl
from jax.experimental.pallas import tpu as pltpu
from jax.experimental.pallas import tpu_sc as plsc

_INFO = plsc.get_sparse_core_info()
_NC, _NS = _INFO.num_cores, _INFO.num_subcores
_NW = _NC * _NS


@functools.lru_cache(maxsize=None)
def _build(vocab, n_embd, n_tokens):
    assert n_tokens % _NW == 0
    per_w = n_tokens // _NW
    chunk = 128
    assert per_w % chunk == 0
    n_chunks = per_w // chunk
    nbuf = 8
    assert n_chunks % nbuf == 0

    mesh = plsc.VectorSubcoreMesh(core_axis_name="c", subcore_axis_name="s")

    @functools.partial(
        pl.kernel,
        out_type=jax.ShapeDtypeStruct((n_tokens, 128), jnp.float32),
        mesh=mesh,
        scratch_types=[
            pltpu.VMEM((n_chunks, chunk), jnp.int32),
            pltpu.VMEM((nbuf, chunk, n_embd), jnp.float32),
        ]
        + [pltpu.SemaphoreType.DMA] * nbuf,
        compiler_params=pltpu.CompilerParams(use_tc_tiling_on_sc=False),
    )
    def emb(table_hbm, idx_hbm, out_hbm, idx_v, bufs, *sems):
        wid = lax.axis_index("s") * _NC + lax.axis_index("c")
        base = wid * per_w
        pltpu.sync_copy(idx_hbm.at[wid], idx_v)

        def start(j, b):
            pltpu.async_copy(table_hbm.at[idx_v.at[j]], bufs.at[b], sems[b])

        def wait(b):
            pltpu.make_async_copy(
                table_hbm.at[pl.ds(0, chunk)], bufs.at[b], sems[b]
            ).wait()

        for b in range(nbuf):
            start(b, b)

        @pl.loop(0, n_chunks, step=nbuf)
        def _(j0):
            for b in range(nbuf):
                j = j0 + b
                wait(b)
                pltpu.sync_copy(
                    bufs.at[b],
                    out_hbm.at[pl.ds(base + j * chunk, chunk), pl.ds(0, n_embd)],
                )
                nxt = j + nbuf

                @pl.when(nxt < n_chunks)
                def _():
                    start(nxt, b)

    return emb, per_w, chunk


def kernel(token_ids, emb_weight):
    b, t = token_ids.shape
    vocab, n_embd = emb_weight.shape
    n_tokens = b * t
    emb, _, chunk = _build(vocab, n_embd, n_tokens)
    idx = token_ids.astype(jnp.int32).reshape(_NW, -1, chunk)
    out_pad = emb(emb_weight, idx)
    return out_pad[:, :n_embd].reshape(b, t, n_embd)
